# Initial kernel scaffold; baseline (speedup 1.0000x reference)
import jax
import jax.numpy as jnp
from jax.experimental import pallas as pl


def kernel(dec_outputs, prev_decOut_tensor, max_length):
    raise NotImplementedError("write your pallas kernel here")


# speedup vs baseline: 130.8926x; 130.8926x over previous
import functools

import numpy as np
import jax
import jax.numpy as jnp
from jax import lax
from jax.experimental import pallas as pl
from jax.experimental.pallas import tpu as pltpu
from jax.experimental.pallas import tpu_sc as plsc

VOCAB = 1000000
NUM_STEPS = 16
TOPK = 50

N_TILES = 25
VPT = 2500
CPT = VPT * 16
NBINS = 4096
CAP = 256
LOCAL_TARGET = 64

_TINY = float(np.finfo(np.float32).tiny)
_I32 = lambda x: jnp.int32(x if x < 2**31 else x - 2**32)


def _np_threefry2x32(k1, k2, x0, x1):
    M = 0xFFFFFFFF
    rot = ((13, 15, 26, 6), (17, 29, 16, 24))
    ks = (k1, k2, (k1 ^ k2 ^ 0x1BD11BDA) & M)
    x = [(x0 + ks[0]) & M, (x1 + ks[1]) & M]

    def rnds(x, rs):
        for r in rs:
            x[0] = (x[0] + x[1]) & M
            x[1] = ((x[1] << r) | (x[1] >> (32 - r))) & M
            x[1] ^= x[0]
        return x

    for i, (ka, kb) in enumerate(((ks[1], ks[2]), (ks[2], ks[0]),
                                  (ks[0], ks[1]), (ks[1], ks[2]),
                                  (ks[2], ks[0]))):
        x = rnds(x, rot[i % 2])
        x[0] = (x[0] + ka) & M
        x[1] = (x[1] + kb + i + 1) & M
    return x[0], x[1]


def _subkey_chain(seed, n):
    key = (seed >> 32, seed & 0xFFFFFFFF)
    out = []
    for _ in range(n):
        nk = _np_threefry2x32(key[0], key[1], 0, 0)
        sk = _np_threefry2x32(key[0], key[1], 0, 1)
        out.append(sk)
        key = nk
    return out


_SUBKEYS = _subkey_chain(42, NUM_STEPS)


def _monotonic_key(bits_i32):
    m = lax.shift_right_arithmetic(bits_i32, 31)
    return lax.bitwise_xor(bits_i32, lax.bitwise_and(m, jnp.int32(0x7FFFFFFF)))


def _sc_topk_body(v_hbm, out_v, out_i, out_c, chunk, hist, cv, ci, cnt_v):
    w = lax.axis_index("c") * 16 + lax.axis_index("s")

    @pl.when(w < N_TILES)
    def _():
        base = w * CPT
        pltpu.sync_copy(v_hbm.at[pl.ds(base, CPT)], chunk)

        lanes = lax.iota(jnp.int32, 16)
        zeros16 = jnp.zeros((16,), jnp.int32)
        ones16 = jnp.ones((16,), jnp.int32)

        def zero_body(b, carry):
            hist[pl.ds(b * 16, 16)] = zeros16
            return carry
        lax.fori_loop(0, NBINS, zero_body, 0)

        def hist_body(j, carry):
            x = chunk[pl.ds(j * 16, 16)]
            key = _monotonic_key(lax.bitcast_convert_type(x, jnp.int32))
            ubin = lax.bitwise_xor(lax.shift_right_logical(key, 20),
                                   jnp.int32(0x800))
            plsc.addupdate_scatter(hist, [ubin * 16 + lanes], ones16)
            return carry
        lax.fori_loop(0, VPT, hist_body, 0)

        def sc_cond(state):
            acc, _ = state
            return acc < LOCAL_TARGET

        def sc_body(state):
            acc, b = state
            b2 = b - 1
            return acc + jnp.sum(hist[pl.ds(b2 * 16, 16)]), b2

        _, tbin = lax.while_loop(sc_cond, sc_body,
                                 (jnp.int32(0), jnp.int32(NBINS)))
        tkey = lax.bitwise_xor(lax.shift_left(tbin, 20), _I32(0x80000000))

        neg_inf16 = jnp.full((16,), -jnp.inf, jnp.float32)
        big16 = jnp.full((16,), 2147483647, jnp.int32)

        def pad_body(j, carry):
            cv[pl.ds(j * 16, 16)] = neg_inf16
            ci[pl.ds(j * 16, 16)] = big16
            return carry
        lax.fori_loop(0, CAP // 16, pad_body, 0)

        def sel_body(j, off):
            x = chunk[pl.ds(j * 16, 16)]
            key = _monotonic_key(lax.bitcast_convert_type(x, jnp.int32))
            msk = key >= tkey
            offc = jnp.minimum(off, CAP - 16)
            plsc.store_compressed(cv.at[pl.ds(offc, 16)], x, mask=msk)
            idxv = (base + j * 16) + lanes
            plsc.store_compressed(ci.at[pl.ds(offc, 16)], idxv, mask=msk)
            return off + jnp.max(plsc.all_reduce_population_count(msk))
        cnt = lax.fori_loop(0, VPT, sel_body, jnp.int32(0))

        cnt_v[...] = jnp.zeros((16,), jnp.int32) + jnp.minimum(cnt, CAP)
        pltpu.sync_copy(cv, out_v.at[w])
        pltpu.sync_copy(ci, out_i.at[w])
        pltpu.sync_copy(cnt_v, out_c.at[w])


@functools.lru_cache(maxsize=None)
def _sc_topk_call():
    return functools.partial(
        pl.kernel,
        out_type=(
            jax.ShapeDtypeStruct((N_TILES, CAP), jnp.float32),
            jax.ShapeDtypeStruct((N_TILES, CAP), jnp.int32),
            jax.ShapeDtypeStruct((N_TILES, 16), jnp.int32),
        ),
        mesh=plsc.VectorSubcoreMesh(core_axis_name="c", subcore_axis_name="s"),
        compiler_params=pltpu.CompilerParams(needs_layout_passes=False),
        scratch_types=[
            pltpu.VMEM((CPT,), jnp.float32),
            pltpu.VMEM((NBINS * 16,), jnp.int32),
            pltpu.VMEM((CAP,), jnp.float32),
            pltpu.VMEM((CAP,), jnp.int32),
            pltpu.VMEM((16,), jnp.int32),
        ],
    )(_sc_topk_body)


def _tf_bits(k1, k2, idx):
    M = 0xFFFFFFFF
    ks = (k1, k2, (k1 ^ k2 ^ 0x1BD11BDA) & M)
    rot = ((13, 15, 26, 6), (17, 29, 16, 24))
    x0 = jnp.full_like(idx, _I32(ks[0]))
    x1 = idx + _I32(ks[1])

    def rnds(x0, x1, rs):
        for r in rs:
            x0 = x0 + x1
            x1 = lax.bitwise_or(lax.shift_left(x1, jnp.int32(r)),
                                lax.shift_right_logical(x1, jnp.int32(32 - r)))
            x1 = lax.bitwise_xor(x1, x0)
        return x0, x1

    for i, (ka, kb) in enumerate(((ks[1], ks[2]), (ks[2], ks[0]),
                                  (ks[0], ks[1]), (ks[1], ks[2]),
                                  (ks[2], ks[0]))):
        x0, x1 = rnds(x0, x1, rot[i % 2])
        x0 = x0 + _I32(ka)
        x1 = x1 + _I32((kb + i + 1) & M)
    return lax.bitwise_xor(x0, x1)


def _tc_sample_body(cv_ref, ci_ref, cc_ref, out_ref):
    vals = cv_ref[...]
    idx = ci_ref[...]
    cnts = cc_ref[...][:, 0:1]
    pos = lax.broadcasted_iota(jnp.int32, (N_TILES, CAP), 1)
    valid = pos < cnts

    NEG = _I32(0x80000000)
    skey = _monotonic_key(lax.bitcast_convert_type(vals, jnp.int32))
    skey = jnp.where(valid, skey, NEG)

    def bs_body(i, P):
        T = lax.bitwise_or(P, lax.shift_left(jnp.int32(1), 31 - i))
        cnt = jnp.sum((skey >= lax.bitwise_xor(T, NEG)).astype(jnp.int32))
        return jnp.where(cnt >= TOPK, T, P)
    P = lax.fori_loop(0, 32, bs_body, jnp.int32(0))
    keep = skey >= lax.bitwise_xor(P, NEG)

    tiny = jnp.float32(_TINY)
    slot = lax.broadcasted_iota(jnp.int32, (1, NUM_STEPS), 1)
    toks = jnp.zeros((1, NUM_STEPS), jnp.int32)
    for t in range(NUM_STEPS):
        k1, k2 = _SUBKEYS[t]
        bits = _tf_bits(k1, k2, idx)
        fb = lax.bitwise_or(lax.shift_right_logical(bits, 9),
                            jnp.int32(0x3F800000))
        f = lax.bitcast_convert_type(fb, jnp.float32) - jnp.float32(1.0)
        u = jnp.maximum(f + tiny, tiny)
        g = -jnp.log(-jnp.log(u))
        score = jnp.where(keep, vals + g, -jnp.inf)
        mx = jnp.max(score)
        win = jnp.min(jnp.where((score == mx) & keep, idx,
                                jnp.int32(2147483647)))
        toks = jnp.where(slot == t, win, toks)
    out_ref[...] = toks


@functools.lru_cache(maxsize=None)
def _tc_sample_call():
    return pl.pallas_call(
        _tc_sample_body,
        out_shape=jax.ShapeDtypeStruct((1, NUM_STEPS), jnp.int32),
    )


BLK = 8192


def _tc_bcast_body(v_ref, o_ref):
    o_ref[...] = jnp.broadcast_to(v_ref[...][None, :], (NUM_STEPS, BLK))


@functools.lru_cache(maxsize=None)
def _tc_bcast_call():
    return pl.pallas_call(
        _tc_bcast_body,
        grid=(pl.cdiv(VOCAB, BLK),),
        in_specs=[pl.BlockSpec((BLK,), lambda i: (i,))],
        out_specs=pl.BlockSpec((NUM_STEPS, BLK), lambda i: (0, i)),
        out_shape=jax.ShapeDtypeStruct((NUM_STEPS, VOCAB), jnp.float32),
    )


def kernel(dec_outputs, prev_decOut_tensor, max_length):
    d8 = dec_outputs.reshape(8, VOCAB)
    v = d8[7]
    cand_v, cand_i, cand_c = _sc_topk_call()(v)
    tokens = _tc_sample_call()(cand_v, cand_i, cand_c).reshape(NUM_STEPS)
    logits = _tc_bcast_call()(v)
    return tokens, logits


# speedup vs baseline: 157.5070x; 1.2033x over previous
import functools

import numpy as np
import jax
import jax.numpy as jnp
from jax import lax
from jax.experimental import pallas as pl
from jax.experimental.pallas import tpu as pltpu
from jax.experimental.pallas import tpu_sc as plsc

VOCAB = 1000000
NUM_STEPS = 16
TOPK = 50

N_TILES = 32
VPT = 1952
VPT_LAST = 62500 - 31 * VPT
CPT = VPT * 16
CPT_LAST = VPT_LAST * 16
UNROLL = 4
NBINS = 4096
CAP = 256
LOCAL_TARGET = 64

_TINY = float(np.finfo(np.float32).tiny)
_I32 = lambda x: jnp.int32(x if x < 2**31 else x - 2**32)


def _np_threefry2x32(k1, k2, x0, x1):
    M = 0xFFFFFFFF
    rot = ((13, 15, 26, 6), (17, 29, 16, 24))
    ks = (k1, k2, (k1 ^ k2 ^ 0x1BD11BDA) & M)
    x = [(x0 + ks[0]) & M, (x1 + ks[1]) & M]

    def rnds(x, rs):
        for r in rs:
            x[0] = (x[0] + x[1]) & M
            x[1] = ((x[1] << r) | (x[1] >> (32 - r))) & M
            x[1] ^= x[0]
        return x

    for i, (ka, kb) in enumerate(((ks[1], ks[2]), (ks[2], ks[0]),
                                  (ks[0], ks[1]), (ks[1], ks[2]),
                                  (ks[2], ks[0]))):
        x = rnds(x, rot[i % 2])
        x[0] = (x[0] + ka) & M
        x[1] = (x[1] + kb + i + 1) & M
    return x[0], x[1]


def _subkey_chain(seed, n):
    key = (seed >> 32, seed & 0xFFFFFFFF)
    out = []
    for _ in range(n):
        nk = _np_threefry2x32(key[0], key[1], 0, 0)
        sk = _np_threefry2x32(key[0], key[1], 0, 1)
        out.append(sk)
        key = nk
    return out


_SUBKEYS = _subkey_chain(42, NUM_STEPS)


def _monotonic_key(bits_i32):
    m = lax.shift_right_arithmetic(bits_i32, 31)
    return lax.bitwise_xor(bits_i32, lax.bitwise_and(m, jnp.int32(0x7FFFFFFF)))


def _sc_topk_body(v_hbm, out_v, out_i, out_c, chunk, hist, cv, ci, cnt_v):
    w = lax.axis_index("c") * 16 + lax.axis_index("s")
    base = w * CPT
    last = w == (N_TILES - 1)

    @pl.when(~last)
    def _():
        pltpu.sync_copy(v_hbm.at[pl.ds(base, CPT)], chunk.at[pl.ds(0, CPT)])

    @pl.when(last)
    def _():
        pltpu.sync_copy(v_hbm.at[pl.ds(31 * CPT, CPT_LAST)], chunk)

    nvec = jnp.where(last, VPT_LAST, VPT)
    lanes = lax.iota(jnp.int32, 16)
    zeros16 = jnp.zeros((16,), jnp.int32)
    ones16 = jnp.ones((16,), jnp.int32)

    def zero_body(b, carry):
        for u in range(8):
            hist[pl.ds((b * 8 + u) * 16, 16)] = zeros16
        return carry
    lax.fori_loop(0, NBINS // 8, zero_body, 0)

    def hist_body(j, kmax):
        for u in range(UNROLL):
            x = chunk[pl.ds((j * UNROLL + u) * 16, 16)]
            key = _monotonic_key(lax.bitcast_convert_type(x, jnp.int32))
            kmax = jnp.maximum(kmax, key)
            ubin = lax.bitwise_xor(lax.shift_right_logical(key, 20),
                                   jnp.int32(0x800))
            plsc.addupdate_scatter(hist, [ubin * 16 + lanes], ones16)
        return kmax
    kmax_v = lax.fori_loop(0, nvec // UNROLL, hist_body,
                           jnp.full((16,), _I32(0x80000000), jnp.int32))
    kmax = jnp.max(kmax_v)
    bmax = lax.bitwise_xor(lax.shift_right_logical(kmax, 20), jnp.int32(0x800))

    def sc_cond(state):
        acc, _ = state
        return acc < LOCAL_TARGET

    def sc_body(state):
        acc, b = state
        b2 = b - 1
        return acc + jnp.sum(hist[pl.ds(b2 * 16, 16)]), b2

    _, tbin = lax.while_loop(sc_cond, sc_body, (jnp.int32(0), bmax + 1))
    tkey = lax.bitwise_xor(lax.shift_left(tbin, 20), _I32(0x80000000))

    neg_inf16 = jnp.full((16,), -jnp.inf, jnp.float32)
    big16 = jnp.full((16,), 2147483647, jnp.int32)

    def pad_body(j, carry):
        cv[pl.ds(j * 16, 16)] = neg_inf16
        ci[pl.ds(j * 16, 16)] = big16
        return carry
    lax.fori_loop(0, CAP // 16, pad_body, 0)

    def sel_body(j, off):
        for u in range(UNROLL):
            x = chunk[pl.ds((j * UNROLL + u) * 16, 16)]
            key = _monotonic_key(lax.bitcast_convert_type(x, jnp.int32))
            msk = key >= tkey
            offc = jnp.minimum(off, CAP - 16)
            plsc.store_compressed(cv.at[pl.ds(offc, 16)], x, mask=msk)
            idxv = (base + (j * UNROLL + u) * 16) + lanes
            plsc.store_compressed(ci.at[pl.ds(offc, 16)], idxv, mask=msk)
            off = off + plsc.all_reduce_population_count(msk)[0]
        return off
    cnt = lax.fori_loop(0, nvec // UNROLL, sel_body, jnp.int32(0))

    cnt_v[...] = jnp.zeros((16,), jnp.int32) + jnp.minimum(cnt, CAP)
    pltpu.sync_copy(cv, out_v.at[w])
    pltpu.sync_copy(ci, out_i.at[w])
    pltpu.sync_copy(cnt_v, out_c.at[w])


@functools.lru_cache(maxsize=None)
def _sc_topk_call():
    return functools.partial(
        pl.kernel,
        out_type=(
            jax.ShapeDtypeStruct((N_TILES, CAP), jnp.float32),
            jax.ShapeDtypeStruct((N_TILES, CAP), jnp.int32),
            jax.ShapeDtypeStruct((N_TILES, 16), jnp.int32),
        ),
        mesh=plsc.VectorSubcoreMesh(core_axis_name="c", subcore_axis_name="s"),
        compiler_params=pltpu.CompilerParams(needs_layout_passes=False),
        scratch_types=[
            pltpu.VMEM((CPT_LAST,), jnp.float32),
            pltpu.VMEM((NBINS * 16,), jnp.int32),
            pltpu.VMEM((CAP,), jnp.float32),
            pltpu.VMEM((CAP,), jnp.int32),
            pltpu.VMEM((16,), jnp.int32),
        ],
    )(_sc_topk_body)


def _tf_bits(k1, k2, idx):
    M = 0xFFFFFFFF
    ks = (k1, k2, (k1 ^ k2 ^ 0x1BD11BDA) & M)
    rot = ((13, 15, 26, 6), (17, 29, 16, 24))
    x0 = jnp.full_like(idx, _I32(ks[0]))
    x1 = idx + _I32(ks[1])

    def rnds(x0, x1, rs):
        for r in rs:
            x0 = x0 + x1
            x1 = lax.bitwise_or(lax.shift_left(x1, jnp.int32(r)),
                                lax.shift_right_logical(x1, jnp.int32(32 - r)))
            x1 = lax.bitwise_xor(x1, x0)
        return x0, x1

    for i, (ka, kb) in enumerate(((ks[1], ks[2]), (ks[2], ks[0]),
                                  (ks[0], ks[1]), (ks[1], ks[2]),
                                  (ks[2], ks[0]))):
        x0, x1 = rnds(x0, x1, rot[i % 2])
        x0 = x0 + _I32(ka)
        x1 = x1 + _I32((kb + i + 1) & M)
    return lax.bitwise_xor(x0, x1)


def _tc_sample_body(cv_ref, ci_ref, cc_ref, out_ref):
    vals = cv_ref[...]
    idx = ci_ref[...]
    cnts = cc_ref[...][:, 0:1]
    pos = lax.broadcasted_iota(jnp.int32, (N_TILES, CAP), 1)
    valid = pos < cnts

    NEG = _I32(0x80000000)
    skey = _monotonic_key(lax.bitcast_convert_type(vals, jnp.int32))
    skey = jnp.where(valid, skey, NEG)

    def bs_body(i, P):
        T = lax.bitwise_or(P, lax.shift_left(jnp.int32(1), 31 - i))
        cnt = jnp.sum((skey >= lax.bitwise_xor(T, NEG)).astype(jnp.int32))
        return jnp.where(cnt >= TOPK, T, P)
    P = lax.fori_loop(0, 32, bs_body, jnp.int32(0))
    keep = skey >= lax.bitwise_xor(P, NEG)

    tiny = jnp.float32(_TINY)
    slot = lax.broadcasted_iota(jnp.int32, (1, NUM_STEPS), 1)
    toks = jnp.zeros((1, NUM_STEPS), jnp.int32)
    for t in range(NUM_STEPS):
        k1, k2 = _SUBKEYS[t]
        bits = _tf_bits(k1, k2, idx)
        fb = lax.bitwise_or(lax.shift_right_logical(bits, 9),
                            jnp.int32(0x3F800000))
        f = lax.bitcast_convert_type(fb, jnp.float32) - jnp.float32(1.0)
        u = jnp.maximum(f + tiny, tiny)
        g = -jnp.log(-jnp.log(u))
        score = jnp.where(keep, vals + g, -jnp.inf)
        mx = jnp.max(score)
        win = jnp.min(jnp.where((score == mx) & keep, idx,
                                jnp.int32(2147483647)))
        toks = jnp.where(slot == t, win, toks)
    out_ref[...] = toks


@functools.lru_cache(maxsize=None)
def _tc_sample_call():
    return pl.pallas_call(
        _tc_sample_body,
        out_shape=jax.ShapeDtypeStruct((1, NUM_STEPS), jnp.int32),
    )


BLK = 8192


def _tc_bcast_body(v_ref, o_ref):
    o_ref[...] = jnp.broadcast_to(v_ref[...][None, :], (NUM_STEPS, BLK))


@functools.lru_cache(maxsize=None)
def _tc_bcast_call():
    return pl.pallas_call(
        _tc_bcast_body,
        grid=(pl.cdiv(VOCAB, BLK),),
        in_specs=[pl.BlockSpec((BLK,), lambda i: (i,))],
        out_specs=pl.BlockSpec((NUM_STEPS, BLK), lambda i: (0, i)),
        out_shape=jax.ShapeDtypeStruct((NUM_STEPS, VOCAB), jnp.float32),
    )


def kernel(dec_outputs, prev_decOut_tensor, max_length):
    d8 = dec_outputs.reshape(8, VOCAB)
    v = d8[7]
    cand_v, cand_i, cand_c = _sc_topk_call()(v)
    tokens = _tc_sample_call()(cand_v, cand_i, cand_c).reshape(NUM_STEPS)
    logits = _tc_bcast_call()(v)
    return tokens, logits


# speedup vs baseline: 194.8387x; 1.2370x over previous
import functools

import numpy as np
import jax
import jax.numpy as jnp
from jax import lax
from jax.experimental import pallas as pl
from jax.experimental.pallas import tpu as pltpu
from jax.experimental.pallas import tpu_sc as plsc

VOCAB = 1000000
NUM_STEPS = 16
TOPK = 50

N_TILES = 32
VPT = 1952
VPT_LAST = 62500 - 31 * VPT
CPT = VPT * 16
CPT_LAST = VPT_LAST * 16
UNROLL = 4
NBINS = 4096
CAP = 256
LOCAL_TARGET = 64

_TINY = float(np.finfo(np.float32).tiny)
_I32 = lambda x: jnp.int32(x if x < 2**31 else x - 2**32)


def _np_threefry2x32(k1, k2, x0, x1):
    M = 0xFFFFFFFF
    rot = ((13, 15, 26, 6), (17, 29, 16, 24))
    ks = (k1, k2, (k1 ^ k2 ^ 0x1BD11BDA) & M)
    x = [(x0 + ks[0]) & M, (x1 + ks[1]) & M]

    def rnds(x, rs):
        for r in rs:
            x[0] = (x[0] + x[1]) & M
            x[1] = ((x[1] << r) | (x[1] >> (32 - r))) & M
            x[1] ^= x[0]
        return x

    for i, (ka, kb) in enumerate(((ks[1], ks[2]), (ks[2], ks[0]),
                                  (ks[0], ks[1]), (ks[1], ks[2]),
                                  (ks[2], ks[0]))):
        x = rnds(x, rot[i % 2])
        x[0] = (x[0] + ka) & M
        x[1] = (x[1] + kb + i + 1) & M
    return x[0], x[1]


def _subkey_chain(seed, n):
    key = (seed >> 32, seed & 0xFFFFFFFF)
    out = []
    for _ in range(n):
        nk = _np_threefry2x32(key[0], key[1], 0, 0)
        sk = _np_threefry2x32(key[0], key[1], 0, 1)
        out.append(sk)
        key = nk
    return out


_SUBKEYS = _subkey_chain(42, NUM_STEPS)


def _monotonic_key(bits_i32):
    m = lax.shift_right_arithmetic(bits_i32, 31)
    return lax.bitwise_xor(bits_i32, lax.bitwise_and(m, jnp.int32(0x7FFFFFFF)))


def _sc_topk_body(v_hbm, out_v, out_i, out_c, out_l, chunk, hist, cv, ci,
                  cnt_v, bsem):
    w = lax.axis_index("c") * 16 + lax.axis_index("s")
    base = w * CPT
    last = w == (N_TILES - 1)

    @pl.when(~last)
    def _():
        pltpu.sync_copy(v_hbm.at[pl.ds(base, CPT)], chunk.at[pl.ds(0, CPT)])
        for r in range(NUM_STEPS):
            pltpu.async_copy(chunk.at[pl.ds(0, CPT)],
                             out_l.at[r, pl.ds(base, CPT)], bsem)

    @pl.when(last)
    def _():
        pltpu.sync_copy(v_hbm.at[pl.ds(31 * CPT, CPT_LAST)], chunk)
        for r in range(NUM_STEPS):
            pltpu.async_copy(chunk, out_l.at[r, pl.ds(31 * CPT, CPT_LAST)],
                             bsem)

    nvec = jnp.where(last, VPT_LAST, VPT)
    lanes = lax.iota(jnp.int32, 16)
    zeros16 = jnp.zeros((16,), jnp.int32)
    ones16 = jnp.ones((16,), jnp.int32)

    def zero_body(b, carry):
        for u in range(8):
            hist[pl.ds((b * 8 + u) * 16, 16)] = zeros16
        return carry
    lax.fori_loop(0, NBINS // 8, zero_body, 0)

    def hist_body(j, kmax):
        for u in range(UNROLL):
            x = chunk[pl.ds((j * UNROLL + u) * 16, 16)]
            key = _monotonic_key(lax.bitcast_convert_type(x, jnp.int32))
            kmax = jnp.maximum(kmax, key)
            ubin = lax.bitwise_xor(lax.shift_right_logical(key, 20),
                                   jnp.int32(0x800))
            plsc.addupdate_scatter(hist, [ubin * 16 + lanes], ones16)
        return kmax
    kmax_v = lax.fori_loop(0, nvec // UNROLL, hist_body,
                           jnp.full((16,), _I32(0x80000000), jnp.int32))
    kmax = jnp.max(kmax_v)
    bmax = lax.bitwise_xor(lax.shift_right_logical(kmax, 20), jnp.int32(0x800))

    def sc_cond(state):
        acc, _ = state
        return acc < LOCAL_TARGET

    def sc_body(state):
        acc, b = state
        b2 = b - 1
        return acc + jnp.sum(hist[pl.ds(b2 * 16, 16)]), b2

    _, tbin = lax.while_loop(sc_cond, sc_body, (jnp.int32(0), bmax + 1))
    tkey = lax.bitwise_xor(lax.shift_left(tbin, 20), _I32(0x80000000))

    neg_inf16 = jnp.full((16,), -jnp.inf, jnp.float32)
    big16 = jnp.full((16,), 2147483647, jnp.int32)

    def pad_body(j, carry):
        cv[pl.ds(j * 16, 16)] = neg_inf16
        ci[pl.ds(j * 16, 16)] = big16
        return carry
    lax.fori_loop(0, CAP // 16, pad_body, 0)

    def sel_body(j, off):
        for u in range(UNROLL):
            x = chunk[pl.ds((j * UNROLL + u) * 16, 16)]
            key = _monotonic_key(lax.bitcast_convert_type(x, jnp.int32))
            msk = key >= tkey
            offc = jnp.minimum(off, CAP - 16)
            plsc.store_compressed(cv.at[pl.ds(offc, 16)], x, mask=msk)
            idxv = (base + (j * UNROLL + u) * 16) + lanes
            plsc.store_compressed(ci.at[pl.ds(offc, 16)], idxv, mask=msk)
            off = off + plsc.all_reduce_population_count(msk)[0]
        return off
    cnt = lax.fori_loop(0, nvec // UNROLL, sel_body, jnp.int32(0))

    cnt_v[...] = jnp.zeros((16,), jnp.int32) + jnp.minimum(cnt, CAP)
    pltpu.sync_copy(cv, out_v.at[w])
    pltpu.sync_copy(ci, out_i.at[w])
    pltpu.sync_copy(cnt_v, out_c.at[w])

    @pl.when(~last)
    def _():
        for r in range(NUM_STEPS):
            pltpu.make_async_copy(chunk.at[pl.ds(0, CPT)],
                                  out_l.at[r, pl.ds(base, CPT)], bsem).wait()

    @pl.when(last)
    def _():
        for r in range(NUM_STEPS):
            pltpu.make_async_copy(chunk,
                                  out_l.at[r, pl.ds(31 * CPT, CPT_LAST)],
                                  bsem).wait()


@functools.lru_cache(maxsize=None)
def _sc_topk_call():
    return functools.partial(
        pl.kernel,
        out_type=(
            jax.ShapeDtypeStruct((N_TILES, CAP), jnp.float32),
            jax.ShapeDtypeStruct((N_TILES, CAP), jnp.int32),
            jax.ShapeDtypeStruct((N_TILES, 16), jnp.int32),
            jax.ShapeDtypeStruct((NUM_STEPS, VOCAB), jnp.float32),
        ),
        mesh=plsc.VectorSubcoreMesh(core_axis_name="c", subcore_axis_name="s"),
        compiler_params=pltpu.CompilerParams(needs_layout_passes=False),
        scratch_types=[
            pltpu.VMEM((CPT_LAST,), jnp.float32),
            pltpu.VMEM((NBINS * 16,), jnp.int32),
            pltpu.VMEM((CAP,), jnp.float32),
            pltpu.VMEM((CAP,), jnp.int32),
            pltpu.VMEM((16,), jnp.int32),
            pltpu.SemaphoreType.DMA,
        ],
    )(_sc_topk_body)


def _tf_bits(k1, k2, idx):
    M = 0xFFFFFFFF
    ks = (k1, k2, (k1 ^ k2 ^ 0x1BD11BDA) & M)
    rot = ((13, 15, 26, 6), (17, 29, 16, 24))
    x0 = jnp.full_like(idx, _I32(ks[0]))
    x1 = idx + _I32(ks[1])

    def rnds(x0, x1, rs):
        for r in rs:
            x0 = x0 + x1
            x1 = lax.bitwise_or(lax.shift_left(x1, jnp.int32(r)),
                                lax.shift_right_logical(x1, jnp.int32(32 - r)))
            x1 = lax.bitwise_xor(x1, x0)
        return x0, x1

    for i, (ka, kb) in enumerate(((ks[1], ks[2]), (ks[2], ks[0]),
                                  (ks[0], ks[1]), (ks[1], ks[2]),
                                  (ks[2], ks[0]))):
        x0, x1 = rnds(x0, x1, rot[i % 2])
        x0 = x0 + _I32(ka)
        x1 = x1 + _I32((kb + i + 1) & M)
    return lax.bitwise_xor(x0, x1)


def _tc_sample_body(cv_ref, ci_ref, cc_ref, out_ref):
    vals = cv_ref[...]
    idx = ci_ref[...]
    cnts = cc_ref[...][:, 0:1]
    pos = lax.broadcasted_iota(jnp.int32, (N_TILES, CAP), 1)
    valid = pos < cnts

    NEG = _I32(0x80000000)
    skey = _monotonic_key(lax.bitcast_convert_type(vals, jnp.int32))
    skey = jnp.where(valid, skey, NEG)

    def bs_body(i, P):
        T = lax.bitwise_or(P, lax.shift_left(jnp.int32(1), 31 - i))
        cnt = jnp.sum((skey >= lax.bitwise_xor(T, NEG)).astype(jnp.int32))
        return jnp.where(cnt >= TOPK, T, P)
    P = lax.fori_loop(0, 32, bs_body, jnp.int32(0))
    keep = skey >= lax.bitwise_xor(P, NEG)

    tiny = jnp.float32(_TINY)
    slot = lax.broadcasted_iota(jnp.int32, (1, NUM_STEPS), 1)
    toks = jnp.zeros((1, NUM_STEPS), jnp.int32)
    for t in range(NUM_STEPS):
        k1, k2 = _SUBKEYS[t]
        bits = _tf_bits(k1, k2, idx)
        fb = lax.bitwise_or(lax.shift_right_logical(bits, 9),
                            jnp.int32(0x3F800000))
        f = lax.bitcast_convert_type(fb, jnp.float32) - jnp.float32(1.0)
        u = jnp.maximum(f + tiny, tiny)
        g = -jnp.log(-jnp.log(u))
        score = jnp.where(keep, vals + g, -jnp.inf)
        mx = jnp.max(score)
        win = jnp.min(jnp.where((score == mx) & keep, idx,
                                jnp.int32(2147483647)))
        toks = jnp.where(slot == t, win, toks)
    out_ref[...] = toks


@functools.lru_cache(maxsize=None)
def _tc_sample_call():
    return pl.pallas_call(
        _tc_sample_body,
        out_shape=jax.ShapeDtypeStruct((1, NUM_STEPS), jnp.int32),
    )


def kernel(dec_outputs, prev_decOut_tensor, max_length):
    v = dec_outputs.reshape(8, VOCAB)[7]
    cand_v, cand_i, cand_c, logits = _sc_topk_call()(v)
    tokens = _tc_sample_call()(cand_v, cand_i, cand_c).reshape(NUM_STEPS)
    return tokens, logits


# speedup vs baseline: 297.3165x; 1.5260x over previous
import functools

import numpy as np
import jax
import jax.numpy as jnp
from jax import lax
from jax.experimental import pallas as pl
from jax.experimental.pallas import tpu as pltpu
from jax.experimental.pallas import tpu_sc as plsc

VOCAB = 1000000
NUM_STEPS = 16
TOPK = 50

N_TILES = 32
VPT = 1952
VPT_LAST = 62500 - 31 * VPT
CPT = VPT * 16
CPT_LAST = VPT_LAST * 16
UNROLL = 4
NBINS = 4096
CAP = 256
LOCAL_TARGET = 64

_TINY = float(np.finfo(np.float32).tiny)
_I32 = lambda x: jnp.int32(x if x < 2**31 else x - 2**32)


def _np_threefry2x32(k1, k2, x0, x1):
    M = 0xFFFFFFFF
    rot = ((13, 15, 26, 6), (17, 29, 16, 24))
    ks = (k1, k2, (k1 ^ k2 ^ 0x1BD11BDA) & M)
    x = [(x0 + ks[0]) & M, (x1 + ks[1]) & M]

    def rnds(x, rs):
        for r in rs:
            x[0] = (x[0] + x[1]) & M
            x[1] = ((x[1] << r) | (x[1] >> (32 - r))) & M
            x[1] ^= x[0]
        return x

    for i, (ka, kb) in enumerate(((ks[1], ks[2]), (ks[2], ks[0]),
                                  (ks[0], ks[1]), (ks[1], ks[2]),
                                  (ks[2], ks[0]))):
        x = rnds(x, rot[i % 2])
        x[0] = (x[0] + ka) & M
        x[1] = (x[1] + kb + i + 1) & M
    return x[0], x[1]


def _subkey_chain(seed, n):
    key = (seed >> 32, seed & 0xFFFFFFFF)
    out = []
    for _ in range(n):
        nk = _np_threefry2x32(key[0], key[1], 0, 0)
        sk = _np_threefry2x32(key[0], key[1], 0, 1)
        out.append(sk)
        key = nk
    return out


_SUBKEYS = _subkey_chain(42, NUM_STEPS)


def _monotonic_key(bits_i32):
    m = lax.shift_right_arithmetic(bits_i32, 31)
    return lax.bitwise_xor(bits_i32, lax.bitwise_and(m, jnp.int32(0x7FFFFFFF)))


def _sc_topk_body(v_hbm, out_v, out_i, out_c, out_l, chunk, hist, cv, ci,
                  cnt_v, bsem):
    w = lax.axis_index("c") * 16 + lax.axis_index("s")
    base = w * CPT
    last = w == (N_TILES - 1)

    @pl.when(~last)
    def _():
        pltpu.sync_copy(v_hbm.at[7, pl.ds(base, CPT)], chunk.at[pl.ds(0, CPT)])
        for r in range(NUM_STEPS):
            pltpu.async_copy(chunk.at[pl.ds(0, CPT)],
                             out_l.at[r, pl.ds(base, CPT)], bsem)

    @pl.when(last)
    def _():
        pltpu.sync_copy(v_hbm.at[7, pl.ds(31 * CPT, CPT_LAST)], chunk)
        for r in range(NUM_STEPS):
            pltpu.async_copy(chunk, out_l.at[r, pl.ds(31 * CPT, CPT_LAST)],
                             bsem)

    nvec = jnp.where(last, VPT_LAST, VPT)
    lanes = lax.iota(jnp.int32, 16)
    zeros16 = jnp.zeros((16,), jnp.int32)
    ones16 = jnp.ones((16,), jnp.int32)

    def zero_body(b, carry):
        for u in range(8):
            hist[pl.ds((b * 8 + u) * 16, 16)] = zeros16
        return carry
    lax.fori_loop(0, NBINS // 8, zero_body, 0)

    def hist_body(j, kmax):
        for u in range(UNROLL):
            x = chunk[pl.ds((j * UNROLL + u) * 16, 16)]
            key = _monotonic_key(lax.bitcast_convert_type(x, jnp.int32))
            kmax = jnp.maximum(kmax, key)
            ubin = lax.bitwise_xor(lax.shift_right_logical(key, 20),
                                   jnp.int32(0x800))
            plsc.addupdate_scatter(hist, [ubin * 16 + lanes], ones16)
        return kmax
    kmax_v = lax.fori_loop(0, nvec // UNROLL, hist_body,
                           jnp.full((16,), _I32(0x80000000), jnp.int32))
    kmax = jnp.max(kmax_v)
    bmax = lax.bitwise_xor(lax.shift_right_logical(kmax, 20), jnp.int32(0x800))

    def sc_cond(state):
        acc, _ = state
        return acc < LOCAL_TARGET

    def sc_body(state):
        acc, b = state
        b2 = b - 1
        return acc + jnp.sum(hist[pl.ds(b2 * 16, 16)]), b2

    _, tbin = lax.while_loop(sc_cond, sc_body, (jnp.int32(0), bmax + 1))
    tkey = lax.bitwise_xor(lax.shift_left(tbin, 20), _I32(0x80000000))

    neg_inf16 = jnp.full((16,), -jnp.inf, jnp.float32)
    big16 = jnp.full((16,), 2147483647, jnp.int32)

    def pad_body(j, carry):
        cv[pl.ds(j * 16, 16)] = neg_inf16
        ci[pl.ds(j * 16, 16)] = big16
        return carry
    lax.fori_loop(0, CAP // 16, pad_body, 0)

    def sel_body(j, off):
        for u in range(UNROLL):
            x = chunk[pl.ds((j * UNROLL + u) * 16, 16)]
            key = _monotonic_key(lax.bitcast_convert_type(x, jnp.int32))
            msk = key >= tkey
            offc = jnp.minimum(off, CAP - 16)
            plsc.store_compressed(cv.at[pl.ds(offc, 16)], x, mask=msk)
            idxv = (base + (j * UNROLL + u) * 16) + lanes
            plsc.store_compressed(ci.at[pl.ds(offc, 16)], idxv, mask=msk)
            off = off + plsc.all_reduce_population_count(msk)[0]
        return off
    cnt = lax.fori_loop(0, nvec // UNROLL, sel_body, jnp.int32(0))

    cnt_v[...] = jnp.zeros((16,), jnp.int32) + jnp.minimum(cnt, CAP)
    pltpu.sync_copy(cv, out_v.at[w])
    pltpu.sync_copy(ci, out_i.at[w])
    pltpu.sync_copy(cnt_v, out_c.at[w])

    @pl.when(~last)
    def _():
        for r in range(NUM_STEPS):
            pltpu.make_async_copy(chunk.at[pl.ds(0, CPT)],
                                  out_l.at[r, pl.ds(base, CPT)], bsem).wait()

    @pl.when(last)
    def _():
        for r in range(NUM_STEPS):
            pltpu.make_async_copy(chunk,
                                  out_l.at[r, pl.ds(31 * CPT, CPT_LAST)],
                                  bsem).wait()


@functools.lru_cache(maxsize=None)
def _sc_topk_call():
    return functools.partial(
        pl.kernel,
        out_type=(
            jax.ShapeDtypeStruct((N_TILES, CAP), jnp.float32),
            jax.ShapeDtypeStruct((N_TILES, CAP), jnp.int32),
            jax.ShapeDtypeStruct((N_TILES, 16), jnp.int32),
            jax.ShapeDtypeStruct((NUM_STEPS, VOCAB), jnp.float32),
        ),
        mesh=plsc.VectorSubcoreMesh(core_axis_name="c", subcore_axis_name="s"),
        compiler_params=pltpu.CompilerParams(needs_layout_passes=False),
        scratch_types=[
            pltpu.VMEM((CPT_LAST,), jnp.float32),
            pltpu.VMEM((NBINS * 16,), jnp.int32),
            pltpu.VMEM((CAP,), jnp.float32),
            pltpu.VMEM((CAP,), jnp.int32),
            pltpu.VMEM((16,), jnp.int32),
            pltpu.SemaphoreType.DMA,
        ],
    )(_sc_topk_body)


def _tf_bits(k1, k2, idx):
    M = 0xFFFFFFFF
    ks = (k1, k2, (k1 ^ k2 ^ 0x1BD11BDA) & M)
    rot = ((13, 15, 26, 6), (17, 29, 16, 24))
    x0 = jnp.full_like(idx, _I32(ks[0]))
    x1 = idx + _I32(ks[1])

    def rnds(x0, x1, rs):
        for r in rs:
            x0 = x0 + x1
            x1 = lax.bitwise_or(lax.shift_left(x1, jnp.int32(r)),
                                lax.shift_right_logical(x1, jnp.int32(32 - r)))
            x1 = lax.bitwise_xor(x1, x0)
        return x0, x1

    for i, (ka, kb) in enumerate(((ks[1], ks[2]), (ks[2], ks[0]),
                                  (ks[0], ks[1]), (ks[1], ks[2]),
                                  (ks[2], ks[0]))):
        x0, x1 = rnds(x0, x1, rot[i % 2])
        x0 = x0 + _I32(ka)
        x1 = x1 + _I32((kb + i + 1) & M)
    return lax.bitwise_xor(x0, x1)


def _tc_sample_body(cv_ref, ci_ref, cc_ref, out_ref):
    vals = cv_ref[...]
    idx = ci_ref[...]
    cnts = cc_ref[...][:, 0:1]
    pos = lax.broadcasted_iota(jnp.int32, (N_TILES, CAP), 1)
    valid = pos < cnts

    NEG = _I32(0x80000000)
    skey = _monotonic_key(lax.bitcast_convert_type(vals, jnp.int32))
    skey = jnp.where(valid, skey, NEG)

    def bs_body(i, P):
        T = lax.bitwise_or(P, lax.shift_left(jnp.int32(1), 31 - i))
        cnt = jnp.sum((skey >= lax.bitwise_xor(T, NEG)).astype(jnp.int32))
        return jnp.where(cnt >= TOPK, T, P)
    P = lax.fori_loop(0, 32, bs_body, jnp.int32(0))
    keep = skey >= lax.bitwise_xor(P, NEG)

    tiny = jnp.float32(_TINY)
    slot = lax.broadcasted_iota(jnp.int32, (1, NUM_STEPS), 1)
    toks = jnp.zeros((1, NUM_STEPS), jnp.int32)
    for t in range(NUM_STEPS):
        k1, k2 = _SUBKEYS[t]
        bits = _tf_bits(k1, k2, idx)
        fb = lax.bitwise_or(lax.shift_right_logical(bits, 9),
                            jnp.int32(0x3F800000))
        f = lax.bitcast_convert_type(fb, jnp.float32) - jnp.float32(1.0)
        u = jnp.maximum(f + tiny, tiny)
        g = -jnp.log(-jnp.log(u))
        score = jnp.where(keep, vals + g, -jnp.inf)
        mx = jnp.max(score)
        win = jnp.min(jnp.where((score == mx) & keep, idx,
                                jnp.int32(2147483647)))
        toks = jnp.where(slot == t, win, toks)
    out_ref[...] = toks


@functools.lru_cache(maxsize=None)
def _tc_sample_call():
    return pl.pallas_call(
        _tc_sample_body,
        out_shape=jax.ShapeDtypeStruct((1, NUM_STEPS), jnp.int32),
    )


def kernel(dec_outputs, prev_decOut_tensor, max_length):
    d8 = dec_outputs.reshape(8, VOCAB)
    cand_v, cand_i, cand_c, logits = _sc_topk_call()(d8)
    tokens = _tc_sample_call()(cand_v, cand_i, cand_c).reshape(NUM_STEPS)
    return tokens, logits


# speedup vs baseline: 314.7006x; 1.0585x over previous
import functools

import numpy as np
import jax
import jax.numpy as jnp
from jax import lax
from jax.experimental import pallas as pl
from jax.experimental.pallas import tpu as pltpu
from jax.experimental.pallas import tpu_sc as plsc

VOCAB = 1000000
NUM_STEPS = 16
TOPK = 50

N_TILES = 32
VPT = 1952
VPT_LAST = 62500 - 31 * VPT
CPT = VPT * 16
CPT_LAST = VPT_LAST * 16
UNROLL = 4
NBINS = 4096
CAP = 256
LOCAL_TARGET = 64

_TINY = float(np.finfo(np.float32).tiny)
_I32 = lambda x: jnp.int32(x if x < 2**31 else x - 2**32)


def _np_threefry2x32(k1, k2, x0, x1):
    M = 0xFFFFFFFF
    rot = ((13, 15, 26, 6), (17, 29, 16, 24))
    ks = (k1, k2, (k1 ^ k2 ^ 0x1BD11BDA) & M)
    x = [(x0 + ks[0]) & M, (x1 + ks[1]) & M]

    def rnds(x, rs):
        for r in rs:
            x[0] = (x[0] + x[1]) & M
            x[1] = ((x[1] << r) | (x[1] >> (32 - r))) & M
            x[1] ^= x[0]
        return x

    for i, (ka, kb) in enumerate(((ks[1], ks[2]), (ks[2], ks[0]),
                                  (ks[0], ks[1]), (ks[1], ks[2]),
                                  (ks[2], ks[0]))):
        x = rnds(x, rot[i % 2])
        x[0] = (x[0] + ka) & M
        x[1] = (x[1] + kb + i + 1) & M
    return x[0], x[1]


def _subkey_chain(seed, n):
    key = (seed >> 32, seed & 0xFFFFFFFF)
    out = []
    for _ in range(n):
        nk = _np_threefry2x32(key[0], key[1], 0, 0)
        sk = _np_threefry2x32(key[0], key[1], 0, 1)
        out.append(sk)
        key = nk
    return out


_SUBKEYS = _subkey_chain(42, NUM_STEPS)


def _monotonic_key(bits_i32):
    m = lax.shift_right_arithmetic(bits_i32, 31)
    return lax.bitwise_xor(bits_i32, lax.bitwise_and(m, jnp.int32(0x7FFFFFFF)))


def _sc_topk_body(v_hbm, out_v, out_i, out_c, out_l, chunk, hist, cv, ci,
                  cnt_v, bsem):
    w = lax.axis_index("c") * 16 + lax.axis_index("s")
    base = w * CPT
    last = w == (N_TILES - 1)

    @pl.when(~last)
    def _():
        pltpu.sync_copy(v_hbm.at[7, pl.ds(base, CPT)], chunk.at[pl.ds(0, CPT)])
        for r in range(NUM_STEPS):
            pltpu.async_copy(chunk.at[pl.ds(0, CPT)],
                             out_l.at[r, pl.ds(base, CPT)], bsem)

    @pl.when(last)
    def _():
        pltpu.sync_copy(v_hbm.at[7, pl.ds(31 * CPT, CPT_LAST)], chunk)
        for r in range(NUM_STEPS):
            pltpu.async_copy(chunk, out_l.at[r, pl.ds(31 * CPT, CPT_LAST)],
                             bsem)

    nvec = jnp.where(last, VPT_LAST, VPT)
    lanes = lax.iota(jnp.int32, 16)
    zeros16 = jnp.zeros((16,), jnp.int32)
    ones16 = jnp.ones((16,), jnp.int32)

    def zero_body(b, carry):
        for u in range(8):
            hist[pl.ds((b * 8 + u) * 16, 16)] = zeros16
        return carry
    lax.fori_loop(0, NBINS // 8, zero_body, 0)

    def hist_body(j, kmax):
        for u in range(UNROLL):
            x = chunk[pl.ds((j * UNROLL + u) * 16, 16)]
            key = _monotonic_key(lax.bitcast_convert_type(x, jnp.int32))
            kmax = jnp.maximum(kmax, key)
            ubin = lax.bitwise_xor(lax.shift_right_logical(key, 20),
                                   jnp.int32(0x800))
            plsc.addupdate_scatter(hist, [ubin * 16 + lanes], ones16)
        return kmax
    kmax_v = lax.fori_loop(0, nvec // UNROLL, hist_body,
                           jnp.full((16,), _I32(0x80000000), jnp.int32))
    kmax = jnp.max(kmax_v)
    bmax = lax.bitwise_xor(lax.shift_right_logical(kmax, 20), jnp.int32(0x800))

    def sc_cond(state):
        acc, _ = state
        return acc < LOCAL_TARGET

    def sc_body(state):
        acc, b = state
        b2 = b - 1
        return acc + jnp.sum(hist[pl.ds(b2 * 16, 16)]), b2

    _, tbin = lax.while_loop(sc_cond, sc_body, (jnp.int32(0), bmax + 1))
    tkey = lax.bitwise_xor(lax.shift_left(tbin, 20), _I32(0x80000000))

    neg_inf16 = jnp.full((16,), -jnp.inf, jnp.float32)
    big16 = jnp.full((16,), 2147483647, jnp.int32)

    def pad_body(j, carry):
        cv[pl.ds(j * 16, 16)] = neg_inf16
        ci[pl.ds(j * 16, 16)] = big16
        return carry
    lax.fori_loop(0, CAP // 16, pad_body, 0)

    def sel_body(j, off):
        xs, ms = [], []
        pops = None
        for u in range(UNROLL):
            x = chunk[pl.ds((j * UNROLL + u) * 16, 16)]
            key = _monotonic_key(lax.bitcast_convert_type(x, jnp.int32))
            m = key >= tkey
            xs.append(x)
            ms.append(m)
            p = plsc.all_reduce_population_count(m)
            pops = p if pops is None else pops + p
        total = pops[0]

        @pl.when(total != 0)
        def _():
            o = off
            for u in range(UNROLL):
                oc = jnp.minimum(o, CAP - 16)
                plsc.store_compressed(cv.at[pl.ds(oc, 16)], xs[u], mask=ms[u])
                idxv = (base + (j * UNROLL + u) * 16) + lanes
                plsc.store_compressed(ci.at[pl.ds(oc, 16)], idxv, mask=ms[u])
                o = o + plsc.all_reduce_population_count(ms[u])[0]
        return off + total
    cnt = lax.fori_loop(0, nvec // UNROLL, sel_body, jnp.int32(0))

    cnt_v[...] = jnp.zeros((16,), jnp.int32) + jnp.minimum(cnt, CAP)
    pltpu.sync_copy(cv, out_v.at[w])
    pltpu.sync_copy(ci, out_i.at[w])
    pltpu.sync_copy(cnt_v, out_c.at[w])

    @pl.when(~last)
    def _():
        for r in range(NUM_STEPS):
            pltpu.make_async_copy(chunk.at[pl.ds(0, CPT)],
                                  out_l.at[r, pl.ds(base, CPT)], bsem).wait()

    @pl.when(last)
    def _():
        for r in range(NUM_STEPS):
            pltpu.make_async_copy(chunk,
                                  out_l.at[r, pl.ds(31 * CPT, CPT_LAST)],
                                  bsem).wait()


@functools.lru_cache(maxsize=None)
def _sc_topk_call():
    return functools.partial(
        pl.kernel,
        out_type=(
            jax.ShapeDtypeStruct((N_TILES, CAP), jnp.float32),
            jax.ShapeDtypeStruct((N_TILES, CAP), jnp.int32),
            jax.ShapeDtypeStruct((N_TILES, 16), jnp.int32),
            jax.ShapeDtypeStruct((NUM_STEPS, VOCAB), jnp.float32),
        ),
        mesh=plsc.VectorSubcoreMesh(core_axis_name="c", subcore_axis_name="s"),
        compiler_params=pltpu.CompilerParams(needs_layout_passes=False),
        scratch_types=[
            pltpu.VMEM((CPT_LAST,), jnp.float32),
            pltpu.VMEM((NBINS * 16,), jnp.int32),
            pltpu.VMEM((CAP,), jnp.float32),
            pltpu.VMEM((CAP,), jnp.int32),
            pltpu.VMEM((16,), jnp.int32),
            pltpu.SemaphoreType.DMA,
        ],
    )(_sc_topk_body)


def _tf_bits(k1, k2, idx):
    M = 0xFFFFFFFF
    ks = (k1, k2, (k1 ^ k2 ^ 0x1BD11BDA) & M)
    rot = ((13, 15, 26, 6), (17, 29, 16, 24))
    x0 = jnp.full_like(idx, _I32(ks[0]))
    x1 = idx + _I32(ks[1])

    def rnds(x0, x1, rs):
        for r in rs:
            x0 = x0 + x1
            x1 = lax.bitwise_or(lax.shift_left(x1, jnp.int32(r)),
                                lax.shift_right_logical(x1, jnp.int32(32 - r)))
            x1 = lax.bitwise_xor(x1, x0)
        return x0, x1

    for i, (ka, kb) in enumerate(((ks[1], ks[2]), (ks[2], ks[0]),
                                  (ks[0], ks[1]), (ks[1], ks[2]),
                                  (ks[2], ks[0]))):
        x0, x1 = rnds(x0, x1, rot[i % 2])
        x0 = x0 + _I32(ka)
        x1 = x1 + _I32((kb + i + 1) & M)
    return lax.bitwise_xor(x0, x1)


def _tc_sample_body(cv_ref, ci_ref, cc_ref, out_ref):
    vals = cv_ref[...]
    idx = ci_ref[...]
    cnts = cc_ref[...][:, 0:1]
    pos = lax.broadcasted_iota(jnp.int32, (N_TILES, CAP), 1)
    valid = pos < cnts

    NEG = _I32(0x80000000)
    skey = _monotonic_key(lax.bitcast_convert_type(vals, jnp.int32))
    skey = jnp.where(valid, skey, NEG)

    def bs_body(i, P):
        T = lax.bitwise_or(P, lax.shift_left(jnp.int32(1), 31 - i))
        cnt = jnp.sum((skey >= lax.bitwise_xor(T, NEG)).astype(jnp.int32))
        return jnp.where(cnt >= TOPK, T, P)
    P = lax.fori_loop(0, 32, bs_body, jnp.int32(0))
    keep = skey >= lax.bitwise_xor(P, NEG)

    tiny = jnp.float32(_TINY)
    slot = lax.broadcasted_iota(jnp.int32, (1, NUM_STEPS), 1)
    toks = jnp.zeros((1, NUM_STEPS), jnp.int32)
    for t in range(NUM_STEPS):
        k1, k2 = _SUBKEYS[t]
        bits = _tf_bits(k1, k2, idx)
        fb = lax.bitwise_or(lax.shift_right_logical(bits, 9),
                            jnp.int32(0x3F800000))
        f = lax.bitcast_convert_type(fb, jnp.float32) - jnp.float32(1.0)
        u = jnp.maximum(f + tiny, tiny)
        g = -jnp.log(-jnp.log(u))
        score = jnp.where(keep, vals + g, -jnp.inf)
        mx = jnp.max(score)
        win = jnp.min(jnp.where((score == mx) & keep, idx,
                                jnp.int32(2147483647)))
        toks = jnp.where(slot == t, win, toks)
    out_ref[...] = toks


@functools.lru_cache(maxsize=None)
def _tc_sample_call():
    return pl.pallas_call(
        _tc_sample_body,
        out_shape=jax.ShapeDtypeStruct((1, NUM_STEPS), jnp.int32),
    )


def kernel(dec_outputs, prev_decOut_tensor, max_length):
    d8 = dec_outputs.reshape(8, VOCAB)
    cand_v, cand_i, cand_c, logits = _sc_topk_call()(d8)
    tokens = _tc_sample_call()(cand_v, cand_i, cand_c).reshape(NUM_STEPS)
    return tokens, logits


# speedup vs baseline: 353.8454x; 1.1244x over previous
import functools

import numpy as np
import jax
import jax.numpy as jnp
from jax import lax
from jax.experimental import pallas as pl
from jax.experimental.pallas import tpu as pltpu
from jax.experimental.pallas import tpu_sc as plsc

VOCAB = 1000000
NUM_STEPS = 16
TOPK = 50

N_TILES = 32
VPT = 1952
VPT_LAST = 62500 - 31 * VPT
CPT = VPT * 16
CPT_LAST = VPT_LAST * 16
UNROLL = 4
NBINS = 4096
CAP = 256
LOCAL_TARGET = 64

_TINY = float(np.finfo(np.float32).tiny)
_I32 = lambda x: jnp.int32(x if x < 2**31 else x - 2**32)


def _np_threefry2x32(k1, k2, x0, x1):
    M = 0xFFFFFFFF
    rot = ((13, 15, 26, 6), (17, 29, 16, 24))
    ks = (k1, k2, (k1 ^ k2 ^ 0x1BD11BDA) & M)
    x = [(x0 + ks[0]) & M, (x1 + ks[1]) & M]

    def rnds(x, rs):
        for r in rs:
            x[0] = (x[0] + x[1]) & M
            x[1] = ((x[1] << r) | (x[1] >> (32 - r))) & M
            x[1] ^= x[0]
        return x

    for i, (ka, kb) in enumerate(((ks[1], ks[2]), (ks[2], ks[0]),
                                  (ks[0], ks[1]), (ks[1], ks[2]),
                                  (ks[2], ks[0]))):
        x = rnds(x, rot[i % 2])
        x[0] = (x[0] + ka) & M
        x[1] = (x[1] + kb + i + 1) & M
    return x[0], x[1]


def _subkey_chain(seed, n):
    key = (seed >> 32, seed & 0xFFFFFFFF)
    out = []
    for _ in range(n):
        nk = _np_threefry2x32(key[0], key[1], 0, 0)
        sk = _np_threefry2x32(key[0], key[1], 0, 1)
        out.append(sk)
        key = nk
    return out


_SUBKEYS = _subkey_chain(42, NUM_STEPS)


def _monotonic_key(bits_i32):
    m = lax.shift_right_arithmetic(bits_i32, 31)
    return lax.bitwise_xor(bits_i32, lax.bitwise_and(m, jnp.int32(0x7FFFFFFF)))


def _sc_topk_body(v_hbm, out_v, out_i, out_c, out_l, chunk, hist, cv, ci,
                  cnt_v, bsem):
    w = lax.axis_index("c") * 16 + lax.axis_index("s")
    base = w * CPT
    last = w == (N_TILES - 1)

    @pl.when(~last)
    def _():
        pltpu.sync_copy(v_hbm.at[7, pl.ds(base, CPT)], chunk.at[pl.ds(0, CPT)])
        for r in range(NUM_STEPS):
            pltpu.async_copy(chunk.at[pl.ds(0, CPT)],
                             out_l.at[r, pl.ds(base, CPT)], bsem)

    @pl.when(last)
    def _():
        pltpu.sync_copy(v_hbm.at[7, pl.ds(31 * CPT, CPT_LAST)], chunk)
        for r in range(NUM_STEPS):
            pltpu.async_copy(chunk, out_l.at[r, pl.ds(31 * CPT, CPT_LAST)],
                             bsem)

    nvec = jnp.where(last, VPT_LAST, VPT)
    lanes = lax.iota(jnp.int32, 16)
    zeros16 = jnp.zeros((16,), jnp.int32)
    ones16 = jnp.ones((16,), jnp.int32)

    def zero_body(b, carry):
        for u in range(8):
            hist[pl.ds((b * 8 + u) * 16, 16)] = zeros16
        return carry
    lax.fori_loop(0, NBINS // 8, zero_body, 0)

    def hist_body(j, kmax):
        for u in range(UNROLL):
            x = chunk[pl.ds((j * UNROLL + u) * 16, 16)]
            key = _monotonic_key(lax.bitcast_convert_type(x, jnp.int32))
            kmax = jnp.maximum(kmax, key)
            ubin = lax.bitwise_xor(lax.shift_right_logical(key, 20),
                                   jnp.int32(0x800))
            plsc.addupdate_scatter(hist, [ubin * 16 + lanes], ones16)
        return kmax
    kmax_v = lax.fori_loop(0, nvec // UNROLL, hist_body,
                           jnp.full((16,), _I32(0x80000000), jnp.int32))
    kmax = jnp.max(kmax_v)
    bmax = lax.bitwise_xor(lax.shift_right_logical(kmax, 20), jnp.int32(0x800))

    def sc_cond(state):
        acc, _ = state
        return acc < LOCAL_TARGET

    def sc_body(state):
        acc, b = state
        b2 = b - 1
        return acc + jnp.sum(hist[pl.ds(b2 * 16, 16)]), b2

    _, tbin = lax.while_loop(sc_cond, sc_body, (jnp.int32(0), bmax + 1))
    tkey = lax.bitwise_xor(lax.shift_left(tbin, 20), _I32(0x80000000))

    neg_inf16 = jnp.full((16,), -jnp.inf, jnp.float32)
    big16 = jnp.full((16,), 2147483647, jnp.int32)

    def pad_body(j, carry):
        cv[pl.ds(j * 16, 16)] = neg_inf16
        ci[pl.ds(j * 16, 16)] = big16
        return carry
    lax.fori_loop(0, CAP // 16, pad_body, 0)

    def _sel_group(vec0, group, off):
        xs, ms = [], []
        pops = None
        for u in range(group):
            x = chunk[pl.ds((vec0 + u) * 16, 16)]
            key = _monotonic_key(lax.bitcast_convert_type(x, jnp.int32))
            m = key >= tkey
            xs.append(x)
            ms.append(m)
            p = plsc.all_reduce_population_count(m)
            pops = p if pops is None else pops + p
        total = pops[0]

        @pl.when(total != 0)
        def _():
            o = off
            for u in range(group):
                oc = jnp.minimum(o, CAP - 16)
                plsc.store_compressed(cv.at[pl.ds(oc, 16)], xs[u], mask=ms[u])
                idxv = (base + (vec0 + u) * 16) + lanes
                plsc.store_compressed(ci.at[pl.ds(oc, 16)], idxv, mask=ms[u])
                o = o + plsc.all_reduce_population_count(ms[u])[0]
        return off + total

    SEL_G = 8
    nvec8 = nvec // SEL_G
    cnt = lax.fori_loop(0, nvec8,
                        lambda j, off: _sel_group(j * SEL_G, SEL_G, off),
                        jnp.int32(0))
    cnt = lax.fori_loop(nvec8 * SEL_G // UNROLL, nvec // UNROLL,
                        lambda j, off: _sel_group(j * UNROLL, UNROLL, off),
                        cnt)

    cnt_v[...] = jnp.zeros((16,), jnp.int32) + jnp.minimum(cnt, CAP)
    pltpu.sync_copy(cv, out_v.at[w])
    pltpu.sync_copy(ci, out_i.at[w])
    pltpu.sync_copy(cnt_v, out_c.at[w])

    @pl.when(~last)
    def _():
        for r in range(NUM_STEPS):
            pltpu.make_async_copy(chunk.at[pl.ds(0, CPT)],
                                  out_l.at[r, pl.ds(base, CPT)], bsem).wait()

    @pl.when(last)
    def _():
        for r in range(NUM_STEPS):
            pltpu.make_async_copy(chunk,
                                  out_l.at[r, pl.ds(31 * CPT, CPT_LAST)],
                                  bsem).wait()


@functools.lru_cache(maxsize=None)
def _sc_topk_call():
    return functools.partial(
        pl.kernel,
        out_type=(
            jax.ShapeDtypeStruct((N_TILES, CAP), jnp.float32),
            jax.ShapeDtypeStruct((N_TILES, CAP), jnp.int32),
            jax.ShapeDtypeStruct((N_TILES, 16), jnp.int32),
            jax.ShapeDtypeStruct((NUM_STEPS, VOCAB), jnp.float32),
        ),
        mesh=plsc.VectorSubcoreMesh(core_axis_name="c", subcore_axis_name="s"),
        compiler_params=pltpu.CompilerParams(needs_layout_passes=False),
        scratch_types=[
            pltpu.VMEM((CPT_LAST,), jnp.float32),
            pltpu.VMEM((NBINS * 16,), jnp.int32),
            pltpu.VMEM((CAP,), jnp.float32),
            pltpu.VMEM((CAP,), jnp.int32),
            pltpu.VMEM((16,), jnp.int32),
            pltpu.SemaphoreType.DMA,
        ],
    )(_sc_topk_body)


def _tf_bits(k1, k2, idx):
    M = 0xFFFFFFFF
    ks = (k1, k2, (k1 ^ k2 ^ 0x1BD11BDA) & M)
    rot = ((13, 15, 26, 6), (17, 29, 16, 24))
    x0 = jnp.full_like(idx, _I32(ks[0]))
    x1 = idx + _I32(ks[1])

    def rnds(x0, x1, rs):
        for r in rs:
            x0 = x0 + x1
            x1 = lax.bitwise_or(lax.shift_left(x1, jnp.int32(r)),
                                lax.shift_right_logical(x1, jnp.int32(32 - r)))
            x1 = lax.bitwise_xor(x1, x0)
        return x0, x1

    for i, (ka, kb) in enumerate(((ks[1], ks[2]), (ks[2], ks[0]),
                                  (ks[0], ks[1]), (ks[1], ks[2]),
                                  (ks[2], ks[0]))):
        x0, x1 = rnds(x0, x1, rot[i % 2])
        x0 = x0 + _I32(ka)
        x1 = x1 + _I32((kb + i + 1) & M)
    return lax.bitwise_xor(x0, x1)


def _tc_sample_body(cv_ref, ci_ref, cc_ref, out_ref):
    vals = cv_ref[...]
    idx = ci_ref[...]
    cnts = cc_ref[...][:, 0:1]
    pos = lax.broadcasted_iota(jnp.int32, (N_TILES, CAP), 1)
    valid = pos < cnts

    NEG = _I32(0x80000000)
    skey = _monotonic_key(lax.bitcast_convert_type(vals, jnp.int32))
    skey = jnp.where(valid, skey, NEG)

    def bs_body(i, P):
        T = lax.bitwise_or(P, lax.shift_left(jnp.int32(1), 31 - i))
        cnt = jnp.sum((skey >= lax.bitwise_xor(T, NEG)).astype(jnp.int32))
        return jnp.where(cnt >= TOPK, T, P)
    P = lax.fori_loop(0, 32, bs_body, jnp.int32(0))
    keep = skey >= lax.bitwise_xor(P, NEG)

    tiny = jnp.float32(_TINY)
    slot = lax.broadcasted_iota(jnp.int32, (1, NUM_STEPS), 1)
    toks = jnp.zeros((1, NUM_STEPS), jnp.int32)
    for t in range(NUM_STEPS):
        k1, k2 = _SUBKEYS[t]
        bits = _tf_bits(k1, k2, idx)
        fb = lax.bitwise_or(lax.shift_right_logical(bits, 9),
                            jnp.int32(0x3F800000))
        f = lax.bitcast_convert_type(fb, jnp.float32) - jnp.float32(1.0)
        u = jnp.maximum(f + tiny, tiny)
        g = -jnp.log(-jnp.log(u))
        score = jnp.where(keep, vals + g, -jnp.inf)
        mx = jnp.max(score)
        win = jnp.min(jnp.where((score == mx) & keep, idx,
                                jnp.int32(2147483647)))
        toks = jnp.where(slot == t, win, toks)
    out_ref[...] = toks


@functools.lru_cache(maxsize=None)
def _tc_sample_call():
    return pl.pallas_call(
        _tc_sample_body,
        out_shape=jax.ShapeDtypeStruct((1, NUM_STEPS), jnp.int32),
    )


def kernel(dec_outputs, prev_decOut_tensor, max_length):
    d8 = dec_outputs.reshape(8, VOCAB)
    cand_v, cand_i, cand_c, logits = _sc_topk_call()(d8)
    tokens = _tc_sample_call()(cand_v, cand_i, cand_c).reshape(NUM_STEPS)
    return tokens, logits


# speedup vs baseline: 366.1691x; 1.0348x over previous
import functools

import numpy as np
import jax
import jax.numpy as jnp
from jax import lax
from jax.experimental import pallas as pl
from jax.experimental.pallas import tpu as pltpu
from jax.experimental.pallas import tpu_sc as plsc

VOCAB = 1000000
NUM_STEPS = 16
TOPK = 50

N_TILES = 32
VPT = 1952
VPT_LAST = 62500 - 31 * VPT
CPT = VPT * 16
CPT_LAST = VPT_LAST * 16
UNROLL = 4
NBINS = 4096
CAP = 256
LOCAL_TARGET = 64

_TINY = float(np.finfo(np.float32).tiny)
_I32 = lambda x: jnp.int32(x if x < 2**31 else x - 2**32)


def _np_threefry2x32(k1, k2, x0, x1):
    M = 0xFFFFFFFF
    rot = ((13, 15, 26, 6), (17, 29, 16, 24))
    ks = (k1, k2, (k1 ^ k2 ^ 0x1BD11BDA) & M)
    x = [(x0 + ks[0]) & M, (x1 + ks[1]) & M]

    def rnds(x, rs):
        for r in rs:
            x[0] = (x[0] + x[1]) & M
            x[1] = ((x[1] << r) | (x[1] >> (32 - r))) & M
            x[1] ^= x[0]
        return x

    for i, (ka, kb) in enumerate(((ks[1], ks[2]), (ks[2], ks[0]),
                                  (ks[0], ks[1]), (ks[1], ks[2]),
                                  (ks[2], ks[0]))):
        x = rnds(x, rot[i % 2])
        x[0] = (x[0] + ka) & M
        x[1] = (x[1] + kb + i + 1) & M
    return x[0], x[1]


def _subkey_chain(seed, n):
    key = (seed >> 32, seed & 0xFFFFFFFF)
    out = []
    for _ in range(n):
        nk = _np_threefry2x32(key[0], key[1], 0, 0)
        sk = _np_threefry2x32(key[0], key[1], 0, 1)
        out.append(sk)
        key = nk
    return out


_SUBKEYS = _subkey_chain(42, NUM_STEPS)


def _monotonic_key(bits_i32):
    m = lax.shift_right_arithmetic(bits_i32, 31)
    return lax.bitwise_xor(bits_i32, lax.bitwise_and(m, jnp.int32(0x7FFFFFFF)))


def _sc_topk_body(v_hbm, out_v, out_i, out_c, out_l, chunk, hist, cv, ci,
                  cnt_v, bsem):
    w = lax.axis_index("c") * 16 + lax.axis_index("s")
    base = w * CPT
    last = w == (N_TILES - 1)

    @pl.when(~last)
    def _():
        pltpu.sync_copy(v_hbm.at[7, pl.ds(base, CPT)], chunk.at[pl.ds(0, CPT)])
        for r in range(NUM_STEPS):
            pltpu.async_copy(chunk.at[pl.ds(0, CPT)],
                             out_l.at[r, pl.ds(base, CPT)], bsem)

    @pl.when(last)
    def _():
        pltpu.sync_copy(v_hbm.at[7, pl.ds(31 * CPT, CPT_LAST)], chunk)
        for r in range(NUM_STEPS):
            pltpu.async_copy(chunk, out_l.at[r, pl.ds(31 * CPT, CPT_LAST)],
                             bsem)

    nvec = jnp.where(last, VPT_LAST, VPT)
    lanes = lax.iota(jnp.int32, 16)
    zeros16 = jnp.zeros((16,), jnp.int32)
    ones16 = jnp.ones((16,), jnp.int32)

    def zero_body(b, carry):
        for u in range(8):
            hist[pl.ds((b * 8 + u) * 16, 16)] = zeros16
        return carry
    lax.fori_loop(0, NBINS // 8, zero_body, 0)

    def hist_body(j, kmax):
        for u in range(UNROLL):
            x = chunk[pl.ds((j * UNROLL + u) * 16, 16)]
            key = _monotonic_key(lax.bitcast_convert_type(x, jnp.int32))
            kmax = jnp.maximum(kmax, key)
            ubin = lax.bitwise_xor(lax.shift_right_logical(key, 20),
                                   jnp.int32(0x800))
            plsc.addupdate_scatter(hist, [ubin * 16 + lanes], ones16)
        return kmax
    kmax_v = lax.fori_loop(0, nvec // UNROLL, hist_body,
                           jnp.full((16,), _I32(0x80000000), jnp.int32))
    kmax = jnp.max(kmax_v)
    bmax = lax.bitwise_xor(lax.shift_right_logical(kmax, 20), jnp.int32(0x800))

    def sc_cond(state):
        acc, _ = state
        return acc < LOCAL_TARGET

    def sc_body(state):
        acc, b = state
        b2 = b - 1
        return acc + jnp.sum(hist[pl.ds(b2 * 16, 16)]), b2

    _, tbin = lax.while_loop(sc_cond, sc_body, (jnp.int32(0), bmax + 1))
    tkey = lax.bitwise_xor(lax.shift_left(tbin, 20), _I32(0x80000000))

    neg_inf16 = jnp.full((16,), -jnp.inf, jnp.float32)
    big16 = jnp.full((16,), 2147483647, jnp.int32)

    def pad_body(j, carry):
        cv[pl.ds(j * 16, 16)] = neg_inf16
        ci[pl.ds(j * 16, 16)] = big16
        return carry
    lax.fori_loop(0, CAP // 16, pad_body, 0)

    def _sel_group(vec0, group, off):
        xs, ms = [], []
        pops = None
        for u in range(group):
            x = chunk[pl.ds((vec0 + u) * 16, 16)]
            key = _monotonic_key(lax.bitcast_convert_type(x, jnp.int32))
            m = key >= tkey
            xs.append(x)
            ms.append(m)
            p = plsc.all_reduce_population_count(m)
            pops = p if pops is None else pops + p
        total = pops[0]

        @pl.when(total != 0)
        def _():
            o = off
            for u in range(group):
                oc = jnp.minimum(o, CAP - 16)
                plsc.store_compressed(cv.at[pl.ds(oc, 16)], xs[u], mask=ms[u])
                idxv = (base + (vec0 + u) * 16) + lanes
                plsc.store_compressed(ci.at[pl.ds(oc, 16)], idxv, mask=ms[u])
                o = o + plsc.all_reduce_population_count(ms[u])[0]
        return off + total

    SEL_G = 16
    nvec8 = nvec // SEL_G
    cnt = lax.fori_loop(0, nvec8,
                        lambda j, off: _sel_group(j * SEL_G, SEL_G, off),
                        jnp.int32(0))
    cnt = lax.fori_loop(nvec8 * SEL_G // UNROLL, nvec // UNROLL,
                        lambda j, off: _sel_group(j * UNROLL, UNROLL, off),
                        cnt)

    cnt_v[...] = jnp.zeros((16,), jnp.int32) + jnp.minimum(cnt, CAP)
    pltpu.sync_copy(cv, out_v.at[w])
    pltpu.sync_copy(ci, out_i.at[w])
    pltpu.sync_copy(cnt_v, out_c.at[w])

    @pl.when(~last)
    def _():
        for r in range(NUM_STEPS):
            pltpu.make_async_copy(chunk.at[pl.ds(0, CPT)],
                                  out_l.at[r, pl.ds(base, CPT)], bsem).wait()

    @pl.when(last)
    def _():
        for r in range(NUM_STEPS):
            pltpu.make_async_copy(chunk,
                                  out_l.at[r, pl.ds(31 * CPT, CPT_LAST)],
                                  bsem).wait()


@functools.lru_cache(maxsize=None)
def _sc_topk_call():
    return functools.partial(
        pl.kernel,
        out_type=(
            jax.ShapeDtypeStruct((N_TILES, CAP), jnp.float32),
            jax.ShapeDtypeStruct((N_TILES, CAP), jnp.int32),
            jax.ShapeDtypeStruct((N_TILES, 16), jnp.int32),
            jax.ShapeDtypeStruct((NUM_STEPS, VOCAB), jnp.float32),
        ),
        mesh=plsc.VectorSubcoreMesh(core_axis_name="c", subcore_axis_name="s"),
        compiler_params=pltpu.CompilerParams(needs_layout_passes=False),
        scratch_types=[
            pltpu.VMEM((CPT_LAST,), jnp.float32),
            pltpu.VMEM((NBINS * 16,), jnp.int32),
            pltpu.VMEM((CAP,), jnp.float32),
            pltpu.VMEM((CAP,), jnp.int32),
            pltpu.VMEM((16,), jnp.int32),
            pltpu.SemaphoreType.DMA,
        ],
    )(_sc_topk_body)


def _tf_bits(k1, k2, idx):
    M = 0xFFFFFFFF
    ks = (k1, k2, (k1 ^ k2 ^ 0x1BD11BDA) & M)
    rot = ((13, 15, 26, 6), (17, 29, 16, 24))
    x0 = jnp.full_like(idx, _I32(ks[0]))
    x1 = idx + _I32(ks[1])

    def rnds(x0, x1, rs):
        for r in rs:
            x0 = x0 + x1
            x1 = lax.bitwise_or(lax.shift_left(x1, jnp.int32(r)),
                                lax.shift_right_logical(x1, jnp.int32(32 - r)))
            x1 = lax.bitwise_xor(x1, x0)
        return x0, x1

    for i, (ka, kb) in enumerate(((ks[1], ks[2]), (ks[2], ks[0]),
                                  (ks[0], ks[1]), (ks[1], ks[2]),
                                  (ks[2], ks[0]))):
        x0, x1 = rnds(x0, x1, rot[i % 2])
        x0 = x0 + _I32(ka)
        x1 = x1 + _I32((kb + i + 1) & M)
    return lax.bitwise_xor(x0, x1)


def _tc_sample_body(cv_ref, ci_ref, cc_ref, out_ref):
    vals = cv_ref[...]
    idx = ci_ref[...]
    cnts = cc_ref[...][:, 0:1]
    pos = lax.broadcasted_iota(jnp.int32, (N_TILES, CAP), 1)
    valid = pos < cnts

    NEG = _I32(0x80000000)
    skey = _monotonic_key(lax.bitcast_convert_type(vals, jnp.int32))
    skey = jnp.where(valid, skey, NEG)

    def bs_body(i, P):
        T = lax.bitwise_or(P, lax.shift_left(jnp.int32(1), 31 - i))
        cnt = jnp.sum((skey >= lax.bitwise_xor(T, NEG)).astype(jnp.int32))
        return jnp.where(cnt >= TOPK, T, P)
    P = lax.fori_loop(0, 32, bs_body, jnp.int32(0))
    keep = skey >= lax.bitwise_xor(P, NEG)

    tiny = jnp.float32(_TINY)
    slot = lax.broadcasted_iota(jnp.int32, (1, NUM_STEPS), 1)
    toks = jnp.zeros((1, NUM_STEPS), jnp.int32)
    for t in range(NUM_STEPS):
        k1, k2 = _SUBKEYS[t]
        bits = _tf_bits(k1, k2, idx)
        fb = lax.bitwise_or(lax.shift_right_logical(bits, 9),
                            jnp.int32(0x3F800000))
        f = lax.bitcast_convert_type(fb, jnp.float32) - jnp.float32(1.0)
        u = jnp.maximum(f + tiny, tiny)
        g = -jnp.log(-jnp.log(u))
        score = jnp.where(keep, vals + g, -jnp.inf)
        mx = jnp.max(score)
        win = jnp.min(jnp.where((score == mx) & keep, idx,
                                jnp.int32(2147483647)))
        toks = jnp.where(slot == t, win, toks)
    out_ref[...] = toks


@functools.lru_cache(maxsize=None)
def _tc_sample_call():
    return pl.pallas_call(
        _tc_sample_body,
        out_shape=jax.ShapeDtypeStruct((1, NUM_STEPS), jnp.int32),
    )


def kernel(dec_outputs, prev_decOut_tensor, max_length):
    d8 = dec_outputs.reshape(8, VOCAB)
    cand_v, cand_i, cand_c, logits = _sc_topk_call()(d8)
    tokens = _tc_sample_call()(cand_v, cand_i, cand_c).reshape(NUM_STEPS)
    return tokens, logits
